# baseline (speedup 1.0000x reference)
import functools

import jax
import jax.numpy as jnp
from jax import lax
from jax.experimental import pallas as pl
from jax.experimental.pallas import tpu as pltpu
from jax.experimental.pallas import tpu_sc as plsc

B = 1024
N = 100000
K = 32
D = 128
ROWS_PER_BLOCK = 8
BIGI = 1 << 30


def _topk_body(z_ref, idx_ref):
    z0 = z_ref[...]
    cols = lax.broadcasted_iota(jnp.int32, z0.shape, 1)
    kidx = lax.broadcasted_iota(jnp.int32, (ROWS_PER_BLOCK, K), 1)

    def step(k, carry):
        z, acc = carry
        m = jnp.max(z, axis=1, keepdims=True)
        j = jnp.min(jnp.where(z == m, cols, BIGI), axis=1, keepdims=True)
        acc = jnp.where(kidx == k, j, acc)
        z = jnp.where(cols == j, -jnp.inf, z)
        return z, acc

    acc0 = jnp.zeros((ROWS_PER_BLOCK, K), jnp.int32)
    _, acc = lax.fori_loop(0, K, step, (z0, acc0))
    idx_ref[...] = acc


def _topk_indices(sp_z):
    grid = (B // ROWS_PER_BLOCK,)
    return pl.pallas_call(
        _topk_body,
        grid=grid,
        in_specs=[pl.BlockSpec((ROWS_PER_BLOCK, N), lambda i: (i, 0))],
        out_specs=pl.BlockSpec((ROWS_PER_BLOCK, K), lambda i: (i, 0)),
        out_shape=jax.ShapeDtypeStruct((B, K), jnp.int32),
    )(sp_z)


def _make_gather():
    info = plsc.get_sparse_core_info()
    nc, ns = info.num_cores, info.num_subcores
    nw = nc * ns
    total = B * K
    b_per_w = total // nw
    chunk = 128
    nchunks = b_per_w // chunk
    mesh = plsc.VectorSubcoreMesh(core_axis_name="c", subcore_axis_name="s")

    @functools.partial(
        pl.kernel,
        mesh=mesh,
        out_type=jax.ShapeDtypeStruct((total, D), jnp.float32),
        scratch_types=[
            pltpu.VMEM((chunk,), jnp.int32),
            pltpu.VMEM((chunk, D), jnp.float32),
            pltpu.SemaphoreType.DMA,
        ],
    )
    def gather(table_hbm, idx_hbm, out_hbm, idx_v, rows_v, sem):
        wid = lax.axis_index("s") * nc + lax.axis_index("c")
        base = wid * b_per_w
        for c in range(nchunks):
            off = base + c * chunk
            pltpu.sync_copy(idx_hbm.at[pl.ds(off, chunk)], idx_v)
            pltpu.async_copy(table_hbm.at[idx_v], rows_v, sem).wait()
            pltpu.sync_copy(rows_v, out_hbm.at[pl.ds(off, chunk)])

    return gather


def _dense_body(ctx_ref, wmap_ref, topw_ref, aligned_ref, out_ref, attn_ref,
                amax_ref):
    ctx = ctx_ref[...]
    wmap = wmap_ref[...]
    aligned = lax.dot_general(ctx, wmap, (((1,), (1,)), ((), ())),
                              preferred_element_type=jnp.float32)
    aligned_ref[...] = aligned
    topw = topw_ref[...]
    r = topw.shape[0]
    sall = lax.dot_general(topw.reshape(r * K, D).astype(jnp.bfloat16),
                           aligned.astype(jnp.bfloat16),
                           (((1,), (1,)), ((), ())),
                           preferred_element_type=jnp.float32)
    rows = lax.broadcasted_iota(jnp.int32, (r * K, r), 0) // K
    colj = lax.broadcasted_iota(jnp.int32, (r * K, r), 1)
    scores = jnp.sum(jnp.where(rows == colj, sall, 0.0), axis=1).reshape(r, K)
    m = jnp.max(scores, axis=1, keepdims=True)
    e = jnp.exp(scores - m)
    attn = e / jnp.sum(e, axis=1, keepdims=True)
    attn_ref[...] = attn
    out_ref[...] = jnp.sum(topw * attn[:, :, None], axis=1)
    ks = lax.broadcasted_iota(jnp.int32, scores.shape, 1)
    amax_ref[...] = jnp.min(jnp.where(scores == m, ks, BIGI), axis=1,
                            keepdims=True)


def _dense(ctx_emb, W_map, topw):
    grid = (B // ROWS_PER_BLOCK,)
    r = ROWS_PER_BLOCK
    return pl.pallas_call(
        _dense_body,
        grid=grid,
        in_specs=[
            pl.BlockSpec((r, D), lambda i: (i, 0)),
            pl.BlockSpec((D, D), lambda i: (0, 0)),
            pl.BlockSpec((r, K, D), lambda i: (i, 0, 0)),
        ],
        out_specs=[
            pl.BlockSpec((r, D), lambda i: (i, 0)),
            pl.BlockSpec((r, D), lambda i: (i, 0)),
            pl.BlockSpec((r, K), lambda i: (i, 0)),
            pl.BlockSpec((r, 1), lambda i: (i, 0)),
        ],
        out_shape=[
            jax.ShapeDtypeStruct((B, D), jnp.float32),
            jax.ShapeDtypeStruct((B, D), jnp.float32),
            jax.ShapeDtypeStruct((B, K), jnp.float32),
            jax.ShapeDtypeStruct((B, 1), jnp.int32),
        ],
    )(ctx_emb, W_map, topw)


def kernel(sp_z, sp_w, ctx_emb, W_map):
    idx = _topk_indices(sp_z)
    topw_flat = _make_gather()(sp_w, idx.reshape(-1))
    topw = topw_flat.reshape(B, K, D)
    aligned, out, attn, amax = _dense(ctx_emb, W_map, topw)
    return (aligned, out, attn[:, :, None], amax[:, 0])


# speedup vs baseline: 3.1215x; 3.1215x over previous
import functools

import jax
import jax.numpy as jnp
from jax import lax
from jax.experimental import pallas as pl
from jax.experimental.pallas import tpu as pltpu
from jax.experimental.pallas import tpu_sc as plsc

B = 1024
N = 100000
K = 32
D = 128
ROWS_PER_BLOCK = 8
BIGI = 1 << 30


NEG_INF = float("-inf")


def _make_topk():
    info = plsc.get_sparse_core_info()
    nc, ns = info.num_cores, info.num_subcores
    nw = nc * ns
    rows_per_w = B // nw
    half = N // 2
    npairs = half // 32
    tail_off = npairs * 32
    nv = half // 16
    ccap = 2048
    mesh = plsc.VectorSubcoreMesh(core_axis_name="c", subcore_axis_name="s")

    @functools.partial(
        pl.kernel,
        mesh=mesh,
        compiler_params=pltpu.CompilerParams(needs_layout_passes=False),
        out_type=jax.ShapeDtypeStruct((B * K,), jnp.int32),
        scratch_types=[
            pltpu.VMEM((half,), jnp.float32),
            pltpu.VMEM((half,), jnp.float32),
            pltpu.VMEM((ccap + 16,), jnp.int32),
            pltpu.VMEM((ccap + 16,), jnp.float32),
            pltpu.VMEM((K,), jnp.int32),
            pltpu.SemaphoreType.DMA,
            pltpu.SemaphoreType.DMA,
        ],
    )
    def topk(z_hbm, out_hbm, buf_a, buf_b, cidx, cval, res, sem_a, sem_b):
        wid = lax.axis_index("s") * nc + lax.axis_index("c")
        row0 = wid * rows_per_w
        lanes = lax.iota(jnp.int32, 16)

        pltpu.async_copy(z_hbm.at[pl.ds(row0 * N, half)], buf_a, sem_a)

        def lane_max(buf, carry):
            def body(i, c):
                ma, mb = c
                v = buf[pl.ds(i * 32, 16)]
                w = buf[pl.ds(i * 32 + 16, 16)]
                return jnp.maximum(ma, v), jnp.maximum(mb, w)

            ma, mb = lax.fori_loop(0, npairs, body, carry, unroll=8)
            return jnp.maximum(ma, buf[pl.ds(tail_off, 16)]), mb

        def collect(buf, t, base, cur0):
            def body(i, cur):
                v = buf[pl.ds(i * 16, 16)]
                m = v >= t
                curc = jnp.minimum(cur, ccap)
                plsc.store_compressed(cidx.at[pl.ds(curc, 16)],
                                      lanes + (base + i * 16), mask=m)
                cnt = plsc.all_reduce_population_count(m)
                return cur + cnt[0]

            return lax.fori_loop(0, nv, body, cur0, unroll=8)

        def gather_half(buf, base, lo, hi):
            def body(j, _):
                off = lo + j * 16
                iv = cidx[pl.ds(off, 16)]
                in_rng = (off + lanes) < hi
                ivc = jnp.clip(iv - base, 0, half - 1)
                vals = plsc.load_gather(buf, [ivc], mask=in_rng)
                cval[pl.ds(off, 16)] = jnp.where(in_rng, vals, NEG_INF)
                return 0

            nj = (hi - lo + 15) // 16
            lax.fori_loop(0, nj, body, 0)

        def extract(c16, row):
            nj = c16 // 16

            def step(k, carry):
                r0, r1, mprev, jprev = carry

                def maxmask(j, vm):
                    off = j * 16
                    v = cval[pl.ds(off, 16)]
                    iv = cidx[pl.ds(off, 16)]
                    v = jnp.where((v == mprev) & (iv == jprev), NEG_INF, v)
                    cval[pl.ds(off, 16)] = v
                    return jnp.maximum(vm, v)

                vm = lax.fori_loop(0, nj, maxmask,
                                   jnp.full((16,), NEG_INF, jnp.float32))
                m = jnp.max(vm)

                def sel(j, im):
                    off = j * 16
                    v = cval[pl.ds(off, 16)]
                    iv = cidx[pl.ds(off, 16)]
                    return jnp.minimum(im, jnp.where(v == m, iv, BIGI))

                im = lax.fori_loop(0, nj, sel,
                                   jnp.full((16,), BIGI, jnp.int32))
                jm = jnp.min(im)
                r0 = jnp.where(lanes == k, jm, r0)
                r1 = jnp.where(lanes == (k - 16), jm, r1)
                return r0, r1, m, jm

            z16 = jnp.zeros((16,), jnp.int32)
            r0, r1, _, _ = lax.fori_loop(
                0, K, step,
                (z16, z16, jnp.float32(NEG_INF), jnp.int32(-1)))
            res[pl.ds(0, 16)] = r0
            res[pl.ds(16, 16)] = r1
            pltpu.sync_copy(res, out_hbm.at[pl.ds(row * K, K)])

        def row_body(r, _):
            row = row0 + r
            pltpu.async_copy(z_hbm.at[pl.ds(row * N + half, half)], buf_b,
                             sem_b)
            pltpu.make_async_copy(z_hbm.at[pl.ds(row * N, half)], buf_a,
                                  sem_a).wait()
            ninf = jnp.full((16,), NEG_INF, jnp.float32)
            ma, mb = lane_max(buf_a, (ninf, ninf))
            pltpu.make_async_copy(z_hbm.at[pl.ds(row * N + half, half)],
                                  buf_b, sem_b).wait()
            ma, mb = lane_max(buf_b, (ma, mb))
            t = jnp.minimum(jnp.min(ma), jnp.min(mb))
            ca = collect(buf_a, t, 0, jnp.int32(0))
            gather_half(buf_a, 0, jnp.int32(0), ca)
            ca16 = ((ca + 15) // 16) * 16

            @pl.when(r + 1 < rows_per_w)
            def _():
                pltpu.async_copy(z_hbm.at[pl.ds((row + 1) * N, half)],
                                 buf_a, sem_a)

            cb = collect(buf_b, t, half, ca16)
            gather_half(buf_b, half, ca16, cb)
            c16 = ((cb + 15) // 16) * 16
            extract(c16, row)
            return 0

        lax.fori_loop(0, rows_per_w, row_body, 0)

    return topk


def _topk_indices(sp_z):
    return _make_topk()(sp_z.reshape(-1)).reshape(B, K)


def _make_gather():
    info = plsc.get_sparse_core_info()
    nc, ns = info.num_cores, info.num_subcores
    nw = nc * ns
    total = B * K
    b_per_w = total // nw
    chunk = 128
    nchunks = b_per_w // chunk
    mesh = plsc.VectorSubcoreMesh(core_axis_name="c", subcore_axis_name="s")

    @functools.partial(
        pl.kernel,
        mesh=mesh,
        compiler_params=pltpu.CompilerParams(needs_layout_passes=False),
        out_type=jax.ShapeDtypeStruct((total, D), jnp.float32),
        scratch_types=[
            pltpu.VMEM((chunk,), jnp.int32),
            pltpu.VMEM((chunk, D), jnp.float32),
            pltpu.SemaphoreType.DMA,
        ],
    )
    def gather(table_hbm, idx_hbm, out_hbm, idx_v, rows_v, sem):
        wid = lax.axis_index("s") * nc + lax.axis_index("c")
        base = wid * b_per_w
        for c in range(nchunks):
            off = base + c * chunk
            pltpu.sync_copy(idx_hbm.at[pl.ds(off, chunk)], idx_v)
            pltpu.async_copy(table_hbm.at[idx_v], rows_v, sem).wait()
            pltpu.sync_copy(rows_v, out_hbm.at[pl.ds(off, chunk)])

    return gather


def _dense_body(ctx_ref, wmap_ref, topw_ref, aligned_ref, out_ref, attn_ref,
                amax_ref):
    ctx = ctx_ref[...]
    wmap = wmap_ref[...]
    aligned = lax.dot_general(ctx, wmap, (((1,), (1,)), ((), ())),
                              preferred_element_type=jnp.float32)
    aligned_ref[...] = aligned
    topw = topw_ref[...]
    r = topw.shape[0]
    sall = lax.dot_general(topw.reshape(r * K, D).astype(jnp.bfloat16),
                           aligned.astype(jnp.bfloat16),
                           (((1,), (1,)), ((), ())),
                           preferred_element_type=jnp.float32)
    rows = lax.broadcasted_iota(jnp.int32, (r * K, r), 0) // K
    colj = lax.broadcasted_iota(jnp.int32, (r * K, r), 1)
    scores = jnp.sum(jnp.where(rows == colj, sall, 0.0), axis=1).reshape(r, K)
    m = jnp.max(scores, axis=1, keepdims=True)
    e = jnp.exp(scores - m)
    attn = e / jnp.sum(e, axis=1, keepdims=True)
    attn_ref[...] = attn
    out_ref[...] = jnp.sum(topw * attn[:, :, None], axis=1)
    ks = lax.broadcasted_iota(jnp.int32, scores.shape, 1)
    amax_ref[...] = jnp.min(jnp.where(scores == m, ks, BIGI), axis=1,
                            keepdims=True)


def _dense(ctx_emb, W_map, topw):
    grid = (B // ROWS_PER_BLOCK,)
    r = ROWS_PER_BLOCK
    return pl.pallas_call(
        _dense_body,
        grid=grid,
        in_specs=[
            pl.BlockSpec((r, D), lambda i: (i, 0)),
            pl.BlockSpec((D, D), lambda i: (0, 0)),
            pl.BlockSpec((r, K, D), lambda i: (i, 0, 0)),
        ],
        out_specs=[
            pl.BlockSpec((r, D), lambda i: (i, 0)),
            pl.BlockSpec((r, D), lambda i: (i, 0)),
            pl.BlockSpec((r, K), lambda i: (i, 0)),
            pl.BlockSpec((r, 1), lambda i: (i, 0)),
        ],
        out_shape=[
            jax.ShapeDtypeStruct((B, D), jnp.float32),
            jax.ShapeDtypeStruct((B, D), jnp.float32),
            jax.ShapeDtypeStruct((B, K), jnp.float32),
            jax.ShapeDtypeStruct((B, 1), jnp.int32),
        ],
    )(ctx_emb, W_map, topw)


def kernel(sp_z, sp_w, ctx_emb, W_map):
    idx = _topk_indices(sp_z)
    topw_flat = _make_gather()(sp_w, idx.reshape(-1))
    topw = topw_flat.reshape(B, K, D)
    aligned, out, attn, amax = _dense(ctx_emb, W_map, topw)
    return (aligned, out, attn[:, :, None], amax[:, 0])


# speedup vs baseline: 6.1316x; 1.9643x over previous
import functools

import jax
import jax.numpy as jnp
from jax import lax
from jax.experimental import pallas as pl
from jax.experimental.pallas import tpu as pltpu
from jax.experimental.pallas import tpu_sc as plsc

B = 1024
N = 100000
K = 32
D = 128
ROWS_PER_BLOCK = 8
BIGI = 1 << 30


NEG_INF = float("-inf")


def _make_topk():
    info = plsc.get_sparse_core_info()
    nc, ns = info.num_cores, info.num_subcores
    nw = nc * ns
    rows_per_w = B // nw
    half = N // 2
    npairs = half // 32
    tail_off = npairs * 32
    nv = half // 16
    ngroups = nv // 16
    ntailv = nv - ngroups * 16
    ccap = 2048
    hcap = 512
    mesh = plsc.VectorSubcoreMesh(core_axis_name="c", subcore_axis_name="s")

    @functools.partial(
        pl.kernel,
        mesh=mesh,
        compiler_params=pltpu.CompilerParams(needs_layout_passes=False),
        out_type=jax.ShapeDtypeStruct((B * K,), jnp.int32),
        scratch_types=[
            pltpu.VMEM((half,), jnp.float32),
            pltpu.VMEM((half,), jnp.float32),
            pltpu.VMEM((ccap + 16,), jnp.int32),
            pltpu.VMEM((ccap + 16,), jnp.float32),
            pltpu.VMEM((hcap + 48,), jnp.int32),
            pltpu.VMEM((K,), jnp.int32),
            pltpu.SemaphoreType.DMA,
            pltpu.SemaphoreType.DMA,
        ],
    )
    def topk(z_hbm, out_hbm, buf_a, buf_b, cidx, cval, hitbuf, res,
             sem_a, sem_b):
        wid = lax.axis_index("s") * nc + lax.axis_index("c")
        row0 = wid * rows_per_w
        lanes = lax.iota(jnp.int32, 16)

        pltpu.async_copy(z_hbm.at[pl.ds(row0 * N, half)], buf_a, sem_a)

        def lane_max(buf, carry):
            def body(i, c):
                ma, mb = c
                v = buf[pl.ds(i * 32, 16)]
                w = buf[pl.ds(i * 32 + 16, 16)]
                return jnp.maximum(ma, v), jnp.maximum(mb, w)

            ma, mb = lax.fori_loop(0, npairs, body, carry, unroll=8)
            return jnp.maximum(ma, buf[pl.ds(tail_off, 16)]), mb

        def detect(buf, t):
            zero16 = jnp.zeros((16,), jnp.int32)

            def group_tail(g, cur, njs):
                hb = zero16
                for j in range(njs):
                    v = buf[pl.ds((g * 16 + j) * 16, 16)]
                    cnt = plsc.all_reduce_population_count(v >= t)
                    hb = jnp.where(lanes == j, cnt, hb)
                m = hb > 0
                curc = jnp.minimum(cur, hcap)
                plsc.store_compressed(hitbuf.at[pl.ds(curc, 16)],
                                      g * 16 + lanes, mask=m)
                return cur + plsc.all_reduce_population_count(m)[0]

            def group(g, cur):
                return group_tail(g, cur, 16)

            cur = lax.fori_loop(0, ngroups, group, jnp.int32(0))
            if ntailv:
                cur = group_tail(jnp.int32(ngroups), cur, ntailv)
            return cur

        def sweep(buf, t, base, nhits, cur0):
            def body(h, cur):
                jv = hitbuf[pl.ds(h, 16)][0]
                v = buf[pl.ds(jv * 16, 16)]
                m = v >= t
                curc = jnp.minimum(cur, ccap)
                plsc.store_compressed(cidx.at[pl.ds(curc, 16)],
                                      jv * 16 + base + lanes, mask=m)
                return cur + plsc.all_reduce_population_count(m)[0]

            return lax.fori_loop(0, nhits, body, cur0)

        def collect(buf, t, base, cur0):
            return sweep(buf, t, base, detect(buf, t), cur0)

        def gather_half(buf, base, lo, hi):
            def body(j, _):
                off = lo + j * 16
                iv = cidx[pl.ds(off, 16)]
                in_rng = (off + lanes) < hi
                ivc = jnp.clip(iv - base, 0, half - 1)
                vals = plsc.load_gather(buf, [ivc], mask=in_rng)
                cval[pl.ds(off, 16)] = jnp.where(in_rng, vals, NEG_INF)
                return 0

            nj = (hi - lo + 15) // 16
            lax.fori_loop(0, nj, body, 0)

        def extract(c16, row):
            nj = c16 // 16

            def step(k, carry):
                r0, r1, mprev, jprev = carry

                def maxmask(j, vm):
                    off = j * 16
                    v = cval[pl.ds(off, 16)]
                    iv = cidx[pl.ds(off, 16)]
                    v = jnp.where((v == mprev) & (iv == jprev), NEG_INF, v)
                    cval[pl.ds(off, 16)] = v
                    return jnp.maximum(vm, v)

                vm = lax.fori_loop(0, nj, maxmask,
                                   jnp.full((16,), NEG_INF, jnp.float32))
                m = jnp.max(vm)

                def sel(j, im):
                    off = j * 16
                    v = cval[pl.ds(off, 16)]
                    iv = cidx[pl.ds(off, 16)]
                    return jnp.minimum(im, jnp.where(v == m, iv, BIGI))

                im = lax.fori_loop(0, nj, sel,
                                   jnp.full((16,), BIGI, jnp.int32))
                jm = jnp.min(im)
                r0 = jnp.where(lanes == k, jm, r0)
                r1 = jnp.where(lanes == (k - 16), jm, r1)
                return r0, r1, m, jm

            z16 = jnp.zeros((16,), jnp.int32)
            r0, r1, _, _ = lax.fori_loop(
                0, K, step,
                (z16, z16, jnp.float32(NEG_INF), jnp.int32(-1)))
            res[pl.ds(0, 16)] = r0
            res[pl.ds(16, 16)] = r1
            pltpu.sync_copy(res, out_hbm.at[pl.ds(row * K, K)])

        def row_body(r, _):
            row = row0 + r
            pltpu.async_copy(z_hbm.at[pl.ds(row * N + half, half)], buf_b,
                             sem_b)
            pltpu.make_async_copy(z_hbm.at[pl.ds(row * N, half)], buf_a,
                                  sem_a).wait()
            ninf = jnp.full((16,), NEG_INF, jnp.float32)
            ma, mb = lane_max(buf_a, (ninf, ninf))
            pltpu.make_async_copy(z_hbm.at[pl.ds(row * N + half, half)],
                                  buf_b, sem_b).wait()
            ma, mb = lane_max(buf_b, (ma, mb))
            t = jnp.minimum(jnp.min(ma), jnp.min(mb))
            ca = collect(buf_a, t, 0, jnp.int32(0))
            gather_half(buf_a, 0, jnp.int32(0), ca)
            ca16 = ((ca + 15) // 16) * 16

            @pl.when(r + 1 < rows_per_w)
            def _():
                pltpu.async_copy(z_hbm.at[pl.ds((row + 1) * N, half)],
                                 buf_a, sem_a)

            cb = collect(buf_b, t, half, ca16)
            gather_half(buf_b, half, ca16, cb)
            c16 = ((cb + 15) // 16) * 16
            extract(c16, row)
            return 0

        lax.fori_loop(0, rows_per_w, row_body, 0)

    return topk


def _topk_indices(sp_z):
    return _make_topk()(sp_z.reshape(-1)).reshape(B, K)


def _make_gather():
    info = plsc.get_sparse_core_info()
    nc, ns = info.num_cores, info.num_subcores
    nw = nc * ns
    total = B * K
    b_per_w = total // nw
    chunk = 128
    nchunks = b_per_w // chunk
    mesh = plsc.VectorSubcoreMesh(core_axis_name="c", subcore_axis_name="s")

    @functools.partial(
        pl.kernel,
        mesh=mesh,
        compiler_params=pltpu.CompilerParams(needs_layout_passes=False),
        out_type=jax.ShapeDtypeStruct((total, D), jnp.float32),
        scratch_types=[
            pltpu.VMEM((chunk,), jnp.int32),
            pltpu.VMEM((chunk, D), jnp.float32),
            pltpu.SemaphoreType.DMA,
        ],
    )
    def gather(table_hbm, idx_hbm, out_hbm, idx_v, rows_v, sem):
        wid = lax.axis_index("s") * nc + lax.axis_index("c")
        base = wid * b_per_w
        for c in range(nchunks):
            off = base + c * chunk
            pltpu.sync_copy(idx_hbm.at[pl.ds(off, chunk)], idx_v)
            pltpu.async_copy(table_hbm.at[idx_v], rows_v, sem).wait()
            pltpu.sync_copy(rows_v, out_hbm.at[pl.ds(off, chunk)])

    return gather


def _dense_body(ctx_ref, wmap_ref, topw_ref, aligned_ref, out_ref, attn_ref,
                amax_ref):
    ctx = ctx_ref[...]
    wmap = wmap_ref[...]
    aligned = lax.dot_general(ctx, wmap, (((1,), (1,)), ((), ())),
                              preferred_element_type=jnp.float32)
    aligned_ref[...] = aligned
    topw = topw_ref[...]
    r = topw.shape[0]
    sall = lax.dot_general(topw.reshape(r * K, D).astype(jnp.bfloat16),
                           aligned.astype(jnp.bfloat16),
                           (((1,), (1,)), ((), ())),
                           preferred_element_type=jnp.float32)
    rows = lax.broadcasted_iota(jnp.int32, (r * K, r), 0) // K
    colj = lax.broadcasted_iota(jnp.int32, (r * K, r), 1)
    scores = jnp.sum(jnp.where(rows == colj, sall, 0.0), axis=1).reshape(r, K)
    m = jnp.max(scores, axis=1, keepdims=True)
    e = jnp.exp(scores - m)
    attn = e / jnp.sum(e, axis=1, keepdims=True)
    attn_ref[...] = attn
    out_ref[...] = jnp.sum(topw * attn[:, :, None], axis=1)
    ks = lax.broadcasted_iota(jnp.int32, scores.shape, 1)
    amax_ref[...] = jnp.min(jnp.where(scores == m, ks, BIGI), axis=1,
                            keepdims=True)


def _dense(ctx_emb, W_map, topw):
    grid = (B // ROWS_PER_BLOCK,)
    r = ROWS_PER_BLOCK
    return pl.pallas_call(
        _dense_body,
        grid=grid,
        in_specs=[
            pl.BlockSpec((r, D), lambda i: (i, 0)),
            pl.BlockSpec((D, D), lambda i: (0, 0)),
            pl.BlockSpec((r, K, D), lambda i: (i, 0, 0)),
        ],
        out_specs=[
            pl.BlockSpec((r, D), lambda i: (i, 0)),
            pl.BlockSpec((r, D), lambda i: (i, 0)),
            pl.BlockSpec((r, K), lambda i: (i, 0)),
            pl.BlockSpec((r, 1), lambda i: (i, 0)),
        ],
        out_shape=[
            jax.ShapeDtypeStruct((B, D), jnp.float32),
            jax.ShapeDtypeStruct((B, D), jnp.float32),
            jax.ShapeDtypeStruct((B, K), jnp.float32),
            jax.ShapeDtypeStruct((B, 1), jnp.int32),
        ],
    )(ctx_emb, W_map, topw)


def kernel(sp_z, sp_w, ctx_emb, W_map):
    idx = _topk_indices(sp_z)
    topw_flat = _make_gather()(sp_w, idx.reshape(-1))
    topw = topw_flat.reshape(B, K, D)
    aligned, out, attn, amax = _dense(ctx_emb, W_map, topw)
    return (aligned, out, attn[:, :, None], amax[:, 0])
